# baseline (speedup 1.0000x reference)
import functools

import jax
import jax.numpy as jnp
from jax import lax
from jax.experimental import pallas as pl
from jax.experimental.pallas import tpu as pltpu
from jax.experimental.pallas import tpu_sc as plsc

DIM = 16
CHUNK = 128


def kernel(table_conf, table_logvar, index_p):
    batch = index_p.shape[0]
    info = plsc.get_sparse_core_info()
    nw = info.num_cores * info.num_subcores
    b_per_w = batch // nw
    n_chunks = b_per_w // CHUNK

    idx3 = index_p.reshape(nw, n_chunks, CHUNK)

    mesh = plsc.VectorSubcoreMesh(core_axis_name="c", subcore_axis_name="s")

    @functools.partial(
        pl.kernel,
        mesh=mesh,
        out_type=(
            jax.ShapeDtypeStruct((batch, DIM), jnp.float32),
            jax.ShapeDtypeStruct((batch, DIM), jnp.float32),
        ),
        scratch_types=[
            pltpu.VMEM((n_chunks, CHUNK), jnp.int32),
            pltpu.VMEM((b_per_w, DIM), jnp.float32),
            pltpu.VMEM((b_per_w, DIM), jnp.float32),
            pltpu.SemaphoreType.DMA,
            pltpu.SemaphoreType.DMA,
        ],
        compiler_params=pltpu.CompilerParams(use_tc_tiling_on_sc=False),
    )
    def _gather2(conf_hbm, logvar_hbm, idx_hbm, z_hbm, zl_hbm,
                 idx_v, rows_a, rows_b, sem_a, sem_b):
        wid = lax.axis_index("s") * info.num_cores + lax.axis_index("c")
        base = wid * b_per_w
        pltpu.sync_copy(idx_hbm.at[wid], idx_v)
        copies = []
        for j in range(n_chunks):
            sl = pl.ds(j * CHUNK, CHUNK)
            copies.append(
                pltpu.async_copy(conf_hbm.at[idx_v.at[j]], rows_a.at[sl], sem_a))
            copies.append(
                pltpu.async_copy(logvar_hbm.at[idx_v.at[j]], rows_b.at[sl], sem_b))
        for c in copies:
            c.wait()
        pltpu.sync_copy(rows_a, z_hbm.at[pl.ds(base, b_per_w)])
        pltpu.sync_copy(rows_b, zl_hbm.at[pl.ds(base, b_per_w)])

    return _gather2(table_conf, table_logvar, idx3)


# speedup vs baseline: 11.3125x; 11.3125x over previous
import functools

import jax
import jax.numpy as jnp
from jax import lax
from jax.experimental import pallas as pl
from jax.experimental.pallas import tpu as pltpu
from jax.experimental.pallas import tpu_sc as plsc

DIM = 16
HALF = 256


def kernel(table_conf, table_logvar, index_p):
    n_rows = table_conf.shape[0]
    batch = index_p.shape[0]
    info = plsc.get_sparse_core_info()
    nw = info.num_cores * info.num_subcores
    b_per_w = batch // nw

    conf_t = table_conf.T.reshape(2, 8, n_rows)
    logvar_t = table_logvar.T.reshape(2, 8, n_rows)
    idx2 = index_p.reshape(nw, b_per_w)

    mesh = plsc.VectorSubcoreMesh(core_axis_name="c", subcore_axis_name="s")

    @functools.partial(
        pl.kernel,
        mesh=mesh,
        out_type=(
            jax.ShapeDtypeStruct((2, 8, batch), jnp.float32),
            jax.ShapeDtypeStruct((2, 8, batch), jnp.float32),
        ),
        scratch_types=[
            pltpu.VMEM((b_per_w,), jnp.int32),
            pltpu.VMEM((2, 8, 8 * HALF), jnp.float32),
            pltpu.VMEM((2, 8, 8 * HALF), jnp.float32),
            pltpu.VMEM((2, 8, b_per_w), jnp.float32),
            pltpu.VMEM((2, 8, b_per_w), jnp.float32),
            pltpu.SemaphoreType.DMA,
            pltpu.SemaphoreType.DMA,
        ],
        compiler_params=pltpu.CompilerParams(needs_layout_passes=False),
    )
    def _gather2(conf_hbm, logvar_hbm, idx_hbm, z_hbm, zl_hbm,
                 idx_v, win_a, win_b, rows_a, rows_b, sem_a, sem_b):
        wid = lax.axis_index("s") * info.num_cores + lax.axis_index("c")
        base = pl.multiple_of(wid * b_per_w, 128)
        pltpu.sync_copy(idx_hbm.at[wid], idx_v)
        lane = lax.iota(jnp.int32, 16)

        for p in range(2):

            def dma_group(g, _):
                vec = idx_v[pl.ds(p * HALF + g * 16, 16)]
                copies = []
                for j in range(16):
                    i = pl.multiple_of(vec[j] & ~7, 8)
                    col = g * 128 + j * 8
                    copies.append(pltpu.async_copy(
                        conf_hbm.at[:, :, pl.ds(i, 8)],
                        win_a.at[:, :, pl.ds(col, 8)], sem_a))
                    copies.append(pltpu.async_copy(
                        logvar_hbm.at[:, :, pl.ds(i, 8)],
                        win_b.at[:, :, pl.ds(col, 8)], sem_b))
                for cp in copies:
                    cp.wait()
                return _

            lax.fori_loop(0, HALF // 16, dma_group, None)

            def select_group(c, _):
                vec = idx_v[pl.ds(p * HALF + c * 16, 16)]
                pos = c * 128 + lane * 8 + (vec & 7)
                out_c = pl.ds(p * HALF + c * 16, 16)
                for t in range(2):
                    for r in range(8):
                        t_vec = jnp.full((16,), t, jnp.int32)
                        r_vec = jnp.full((16,), r, jnp.int32)
                        rows_a[t, r, out_c] = plsc.load_gather(
                            win_a, [t_vec, r_vec, pos])
                        rows_b[t, r, out_c] = plsc.load_gather(
                            win_b, [t_vec, r_vec, pos])
                return _

            lax.fori_loop(0, HALF // 16, select_group, None)

        out_sl = pl.ds(base, b_per_w)
        pltpu.sync_copy(rows_a, z_hbm.at[:, :, out_sl])
        pltpu.sync_copy(rows_b, zl_hbm.at[:, :, out_sl])

    zt, zlt = _gather2(conf_t, logvar_t, idx2)
    z = zt.reshape(DIM, batch).T
    zl = zlt.reshape(DIM, batch).T
    return (z, zl)
